# baseline (speedup 1.0000x reference)
import functools

import jax
import jax.numpy as jnp
from jax import lax
from jax.experimental import pallas as pl
from jax.experimental.pallas import tpu as pltpu
from jax.experimental.pallas import tpu_sc as plsc

N, D, H, E, K = 4096, 1024, 1024, 8, 2
M = N * K
BLK = 256
P = M + E * BLK
NB = P // BLK

NC, NS = 2, 16
NW = NC * NS

TOKS_W = N // NW
CHUNK = 32


def _routing(top_idx, top_gates):
    ef = top_idx.reshape(-1).astype(jnp.int32)
    oh = (ef[:, None] == jnp.arange(E, dtype=jnp.int32)[None, :]).astype(jnp.int32)
    cum = jnp.cumsum(oh, axis=0)
    rank = (oh * cum).sum(axis=1) - 1
    counts = cum[-1]
    padded = ((counts + BLK - 1) // BLK) * BLK
    ends = jnp.cumsum(padded)
    starts = ends - padded
    pos = (oh * starts[None, :]).sum(axis=1) + rank
    blk_starts = jnp.arange(NB, dtype=jnp.int32) * BLK
    be = jnp.minimum((ends[None, :] <= blk_starts[:, None]).sum(axis=1), E - 1)
    pos2 = pos.reshape(N, K)
    return be.astype(jnp.int32), pos2[:, 0], pos2[:, 1]


@functools.cache
def _make_sc_kernels():
    mesh = plsc.VectorSubcoreMesh(core_axis_name="c", subcore_axis_name="s")

    @functools.partial(
        pl.kernel,
        out_type=jax.ShapeDtypeStruct((P, D), jnp.float32),
        mesh=mesh,
        scratch_types=[
            pltpu.VMEM((CHUNK,), jnp.int32),
            pltpu.VMEM((CHUNK,), jnp.int32),
            pltpu.VMEM((CHUNK, D), jnp.float32),
            pltpu.SemaphoreType.DMA,
        ],
    )
    def sc_dispatch(x_hbm, pos0_hbm, pos1_hbm, xs_hbm, p0_v, p1_v, xbuf_v, sem):
        wid = lax.axis_index("s") * NC + lax.axis_index("c")
        for ch in range(TOKS_W // CHUNK):
            base = wid * TOKS_W + ch * CHUNK
            pltpu.sync_copy(x_hbm.at[pl.ds(base, CHUNK)], xbuf_v)
            pltpu.sync_copy(pos0_hbm.at[pl.ds(base, CHUNK)], p0_v)
            pltpu.sync_copy(pos1_hbm.at[pl.ds(base, CHUNK)], p1_v)
            pltpu.async_copy(xbuf_v, xs_hbm.at[p0_v], sem).wait()
            pltpu.async_copy(xbuf_v, xs_hbm.at[p1_v], sem).wait()

    @functools.partial(
        pl.kernel,
        out_type=jax.ShapeDtypeStruct((N, H), jnp.float32),
        mesh=mesh,
        scratch_types=[
            pltpu.VMEM((CHUNK,), jnp.int32),
            pltpu.VMEM((CHUNK,), jnp.int32),
            pltpu.VMEM((CHUNK,), jnp.float32),
            pltpu.VMEM((CHUNK,), jnp.float32),
            pltpu.VMEM((CHUNK, H), jnp.float32),
            pltpu.VMEM((CHUNK, H), jnp.float32),
            pltpu.SemaphoreType.DMA,
        ],
    )
    def sc_combine(rows_hbm, pos0_hbm, pos1_hbm, g0_hbm, g1_hbm, y_hbm,
                   p0_v, p1_v, g0_v, g1_v, r0_v, r1_v, sem):
        wid = lax.axis_index("s") * NC + lax.axis_index("c")
        for ch in range(TOKS_W // CHUNK):
            base = wid * TOKS_W + ch * CHUNK
            pltpu.sync_copy(pos0_hbm.at[pl.ds(base, CHUNK)], p0_v)
            pltpu.sync_copy(pos1_hbm.at[pl.ds(base, CHUNK)], p1_v)
            pltpu.sync_copy(g0_hbm.at[pl.ds(base, CHUNK)], g0_v)
            pltpu.sync_copy(g1_hbm.at[pl.ds(base, CHUNK)], g1_v)
            pltpu.async_copy(rows_hbm.at[p0_v], r0_v, sem).wait()
            pltpu.async_copy(rows_hbm.at[p1_v], r1_v, sem).wait()

            def _combine_group(tg, _):
                g0vec = g0_v[pl.ds(tg * 16, 16)]
                g1vec = g1_v[pl.ds(tg * 16, 16)]
                g0s = [g0vec[i] for i in range(16)]
                g1s = [g1vec[i] for i in range(16)]

                def _combine_col(c, _):
                    sl = pl.ds(c * 16, 16)
                    for i in range(16):
                        t = tg * 16 + i
                        r0_v[t, sl] = g0s[i] * r0_v[t, sl] + g1s[i] * r1_v[t, sl]
                    return 0

                lax.fori_loop(0, H // 16, _combine_col, 0)
                return 0

            lax.fori_loop(0, CHUNK // 16, _combine_group, 0)
            pltpu.sync_copy(r0_v, y_hbm.at[pl.ds(base, CHUNK)])

    return sc_dispatch, sc_combine


def _group_mm_body(be_ref, xs_ref, w_ref, b_ref, o_ref):
    acc = jnp.dot(xs_ref[...].astype(jnp.bfloat16), w_ref[0],
                  preferred_element_type=jnp.float32)
    o_ref[...] = acc + b_ref[0]


@jax.jit
def _tc_group_mm(block_expert, x_sorted, w_bf, bias3):
    grid_spec = pltpu.PrefetchScalarGridSpec(
        num_scalar_prefetch=1,
        grid=(NB,),
        in_specs=[
            pl.BlockSpec((BLK, D), lambda i, be: (i, 0)),
            pl.BlockSpec((1, D, H), lambda i, be: (be[i], 0, 0)),
            pl.BlockSpec((1, 1, H), lambda i, be: (be[i], 0, 0)),
        ],
        out_specs=pl.BlockSpec((BLK, H), lambda i, be: (i, 0)),
    )
    return pl.pallas_call(
        _group_mm_body,
        grid_spec=grid_spec,
        out_shape=jax.ShapeDtypeStruct((P, H), jnp.float32),
    )(block_expert, x_sorted, w_bf, bias3)


def kernel(x, w_gate, w_noise, expert_w, expert_b):
    clean_logits = x @ w_gate
    raw_noise_stddev = x @ w_noise
    noise_stddev = jax.nn.softplus(raw_noise_stddev) + 1e-2
    noise = jax.random.normal(jax.random.key(42), clean_logits.shape, dtype=clean_logits.dtype)
    logits = clean_logits + noise * noise_stddev
    top_vals, top_idx = jax.lax.top_k(logits, K)
    top_gates = jax.nn.softmax(top_vals, axis=-1)

    block_expert, pos0, pos1 = _routing(top_idx, top_gates)
    g0 = top_gates[:, 0]
    g1 = top_gates[:, 1]

    sc_dispatch, sc_combine = _make_sc_kernels()
    x_sorted = sc_dispatch(x, pos0, pos1)
    w_bf = expert_w.astype(jnp.bfloat16)
    out_sorted = _tc_group_mm(block_expert, x_sorted, w_bf, expert_b[:, None, :])
    return sc_combine(out_sorted, pos0, pos1, g0, g1)
